# baseline (speedup 1.0000x reference)
import functools

import jax
import jax.numpy as jnp
from jax import lax
from jax.experimental import pallas as pl
from jax.experimental.pallas import tpu as pltpu
from jax.experimental.pallas import tpu_sc as plsc

LATENT_DIM = 16
BATCH = 16384
REG_BIAS = 0.00013
REG_LATENT = 0.00018

NC, NS, L = 2, 16, 16
NW = NC * NS
BPW = BATCH // NW
CHUNK = 128
NCHUNK = BPW // CHUNK
NBLK = BPW // L

_mesh = plsc.VectorSubcoreMesh(
    core_axis_name="c", subcore_axis_name="s", num_cores=NC, num_subcores=NS
)


@functools.partial(
    pl.kernel,
    out_type=[
        jax.ShapeDtypeStruct((BATCH, L), jnp.float32),
        jax.ShapeDtypeStruct((BATCH,), jnp.float32),
        jax.ShapeDtypeStruct((BATCH,), jnp.float32),
        jax.ShapeDtypeStruct((NW, L), jnp.float32),
    ],
    mesh=_mesh,
    compiler_params=pltpu.CompilerParams(use_tc_tiling_on_sc=False),
    scratch_types=[
        pltpu.VMEM((NCHUNK, CHUNK), jnp.int32),
        pltpu.VMEM((NCHUNK, CHUNK), jnp.int32),
        pltpu.VMEM((NCHUNK, CHUNK), jnp.int32),
        pltpu.VMEM((BPW, L), jnp.float32),
        pltpu.VMEM((BPW, L), jnp.float32),
        pltpu.VMEM((BPW, L), jnp.float32),
        pltpu.VMEM((BPW, L), jnp.float32),
        pltpu.VMEM((BPW,), jnp.float32),
        pltpu.VMEM((BPW,), jnp.float32),
        pltpu.VMEM((L,), jnp.float32),
        pltpu.SemaphoreType.DMA,
    ],
)
def _sc_gather(uf, itf, ib, ui, pi, ni,
               prod_out, pb_out, nb_out, se_out,
               uidx, pidx, nidx, urows, prows, nrows, prodv, pb, nb,
               sev, sem):
    wid = lax.axis_index("s") * NC + lax.axis_index("c")
    base = wid * BPW

    pltpu.sync_copy(ui.at[wid], uidx)
    pltpu.sync_copy(pi.at[wid], pidx)
    pltpu.sync_copy(ni.at[wid], nidx)

    copies = []
    for c in range(NCHUNK):
        sl = pl.ds(c * CHUNK, CHUNK)
        copies.append(pltpu.async_copy(uf.at[uidx.at[c]], urows.at[sl, :], sem))
        copies.append(pltpu.async_copy(itf.at[pidx.at[c]], prows.at[sl, :], sem))
        copies.append(pltpu.async_copy(itf.at[nidx.at[c]], nrows.at[sl, :], sem))
        copies.append(pltpu.async_copy(ib.at[pidx.at[c]], pb.at[sl], sem))
        copies.append(pltpu.async_copy(ib.at[nidx.at[c]], nb.at[sl], sem))
    for cp in copies:
        cp.wait()

    def row(i, sq):
        u = urows[i, :]
        p = prows[i, :]
        n = nrows[i, :]
        prodv[i, :] = u * (p - n)
        return sq + u * u + p * p + n * n

    sq = lax.fori_loop(0, BPW, row, jnp.zeros((L,), jnp.float32))
    sev[...] = sq

    pltpu.sync_copy(prodv, prod_out.at[pl.ds(base, BPW), :])
    pltpu.sync_copy(pb, pb_out.at[pl.ds(base, BPW)])
    pltpu.sync_copy(nb, nb_out.at[pl.ds(base, BPW)])
    pltpu.sync_copy(sev, se_out.at[wid])


def _tc_loss_body(prod_ref, pb_ref, nb_ref, se_ref, o_ref):
    pbv = pb_ref[...]
    nbv = nb_ref[...]
    x = jnp.sum(prod_ref[...], axis=1) + pbv - nbv
    s = 1.0 / (1.0 + jnp.exp(-x)) + 1e-10
    loss = -jnp.sum(jnp.log(s)) / BATCH
    reg = REG_BIAS * (
        jnp.sqrt(jnp.sum(pbv * pbv)) + jnp.sqrt(jnp.sum(nbv * nbv))
    ) * 0.5
    reg = reg + REG_LATENT * jnp.sum(se_ref[...])
    o_ref[...] = jnp.broadcast_to(loss + reg, (1, 1))


_tc_loss = pl.pallas_call(
    _tc_loss_body,
    out_shape=jax.ShapeDtypeStruct((1, 1), jnp.float32),
)


def kernel(user_factors, item_factors, item_bias,
           user_indices, pos_item_indices, neg_item_indices):
    ui = user_indices.astype(jnp.int32).reshape(NW, NCHUNK, CHUNK)
    pi = pos_item_indices.astype(jnp.int32).reshape(NW, NCHUNK, CHUNK)
    ni = neg_item_indices.astype(jnp.int32).reshape(NW, NCHUNK, CHUNK)
    prod, pb, nb, se = _sc_gather(user_factors, item_factors,
                                  item_bias.reshape(-1), ui, pi, ni)
    out = _tc_loss(prod, pb, nb, se)
    return out[0, 0]
